# baseline (speedup 1.0000x reference)
import functools

import jax
import jax.numpy as jnp
from jax import lax
from jax.experimental import pallas as pl
from jax.experimental.pallas import tpu as pltpu
from jax.experimental.pallas import tpu_sc as plsc

N = 10000
NP = 10240
E = 160000
EP = 163840
NW = 32
CH = 128
NCH = EP // (NW * CH)
H = 32
ED = 16
B = 64
OUT = 128

@functools.cache
def _mesh():
    return plsc.VectorSubcoreMesh(core_axis_name="c", subcore_axis_name="s")


_SC_PARAMS = pltpu.CompilerParams(use_tc_tiling_on_sc=False)


def _sc_gather(table, idx3):
    @functools.partial(
        pl.kernel, mesh=_mesh(), compiler_params=_SC_PARAMS,
        out_type=jax.ShapeDtypeStruct((EP, H), jnp.float32),
        scratch_types=[
            pltpu.VMEM((NCH, CH), jnp.int32),
            pltpu.VMEM((CH, H), jnp.float32),
            pltpu.SemaphoreType.DMA,
        ],
    )
    def k(table_hbm, idx_hbm, out_hbm, idx_v, buf, sem):
        wid = lax.axis_index("s") * 2 + lax.axis_index("c")
        base = wid * (NCH * CH)
        pltpu.sync_copy(idx_hbm.at[wid], idx_v)

        @pl.loop(0, NCH)
        def _(j):
            pltpu.async_copy(table_hbm.at[idx_v.at[j]], buf, sem).wait()
            pltpu.sync_copy(buf, out_hbm.at[pl.ds(base + j * CH, CH)])

    return k(table, idx3)


def _sc_scatter_add(vals, idx3, zeros):
    @functools.partial(
        pl.kernel, mesh=_mesh(), compiler_params=_SC_PARAMS,
        out_type=jax.ShapeDtypeStruct((2, NP, H), jnp.float32),
        scratch_types=[
            pltpu.VMEM((NCH, CH), jnp.int32),
            pltpu.VMEM((CH, H), jnp.float32),
            pltpu.VMEM_SHARED((NP, H), jnp.float32),
            pltpu.SemaphoreType.DMA,
        ],
    )
    def k(v_hbm, idx_hbm, z_hbm, out_hbm, idx_v, buf, shared, sem):
        cid = lax.axis_index("c")
        sid = lax.axis_index("s")
        wid = sid * 2 + cid
        rows = NP // 16
        pltpu.sync_copy(z_hbm.at[pl.ds(sid * rows, rows)],
                        shared.at[pl.ds(sid * rows, rows)])
        pltpu.sync_copy(idx_hbm.at[wid], idx_v)
        plsc.subcore_barrier()

        @pl.loop(0, NCH)
        def _(j):
            pltpu.async_copy(
                v_hbm.at[pl.ds(wid * (NCH * CH) + j * CH, CH)], buf, sem
            ).wait()
            pltpu.sync_copy(buf, shared.at[idx_v.at[j]], add=True)

        plsc.subcore_barrier()
        pltpu.sync_copy(shared.at[pl.ds(sid * rows, rows)],
                        out_hbm.at[cid, pl.ds(sid * rows, rows)])

    return k(vals, idx3, zeros)


def _sc_degree(idx3, ones, zeros):
    @functools.partial(
        pl.kernel, mesh=_mesh(), compiler_params=_SC_PARAMS,
        out_type=jax.ShapeDtypeStruct((2, NP, 16), jnp.float32),
        scratch_types=[
            pltpu.VMEM((NCH, CH), jnp.int32),
            pltpu.VMEM((CH, 16), jnp.float32),
            pltpu.VMEM_SHARED((NP, 16), jnp.float32),
            pltpu.SemaphoreType.DMA,
        ],
    )
    def k(idx_hbm, ones_hbm, z_hbm, out_hbm, idx_v, buf, shared, sem):
        cid = lax.axis_index("c")
        sid = lax.axis_index("s")
        wid = sid * 2 + cid
        rows = NP // 16
        pltpu.sync_copy(z_hbm.at[pl.ds(sid * rows, rows)],
                        shared.at[pl.ds(sid * rows, rows)])
        pltpu.sync_copy(idx_hbm.at[wid], idx_v)
        pltpu.sync_copy(ones_hbm, buf)
        plsc.subcore_barrier()

        @pl.loop(0, NCH)
        def _(j):
            pltpu.sync_copy(buf, shared.at[idx_v.at[j]], add=True)

        plsc.subcore_barrier()
        pltpu.sync_copy(shared.at[pl.ds(sid * rows, rows)],
                        out_hbm.at[cid, pl.ds(sid * rows, rows)])

    return k(idx3, ones, zeros)


def _tc_node_init(xp, W0, b0b):
    def body(x_ref, w_ref, b_ref, o_ref):
        o_ref[...] = jnp.maximum(
            jnp.dot(x_ref[...], w_ref[...], preferred_element_type=jnp.float32)
            + b_ref[0:1, :], 0.0)

    return pl.pallas_call(
        body, out_shape=jax.ShapeDtypeStruct((NP, H), jnp.float32)
    )(xp, W0, b0b)


_TE = 512


def _tc_msg(eap, xj, Wn1, bn1b, Wn2, bn2b):
    def body(ea_ref, xj_ref, w1_ref, b1_ref, w2_ref, b2_ref, o_ref):
        f = jnp.maximum(
            jnp.dot(ea_ref[...], w1_ref[...], preferred_element_type=jnp.float32)
            + b1_ref[0:1, :], 0.0)
        U = jnp.dot(f, w2_ref[...], preferred_element_type=jnp.float32) \
            + b2_ref[0:1, :]
        xj = xj_ref[...]
        acc = xj[:, 0:1] * U[:, 0:H]
        for h in range(1, H):
            acc = acc + xj[:, h:h + 1] * U[:, h * H:(h + 1) * H]
        o_ref[...] = acc

    return pl.pallas_call(
        body,
        grid=(EP // _TE,),
        in_specs=[
            pl.BlockSpec((_TE, ED), lambda i: (i, 0)),
            pl.BlockSpec((_TE, H), lambda i: (i, 0)),
            pl.BlockSpec((ED, 128), lambda i: (0, 0)),
            pl.BlockSpec((8, 128), lambda i: (0, 0)),
            pl.BlockSpec((128, H * H), lambda i: (0, 0)),
            pl.BlockSpec((8, H * H), lambda i: (0, 0)),
        ],
        out_specs=pl.BlockSpec((_TE, H), lambda i: (i, 0)),
        out_shape=jax.ShapeDtypeStruct((EP, H), jnp.float32),
    )(eap, xj, Wn1, bn1b, Wn2, bn2b)


def _tc_update(p0, p1, d0, d1, out, h, Wroot, bconvb, WihT, WhhT, bihb, bhhb):
    def body(p0_ref, p1_ref, d0_ref, d1_ref, o_ref, h_ref, wr_ref, bc_ref,
             wi_ref, wh_ref, bi_ref, bh_ref, new_ref):
        deg = jnp.maximum(d0_ref[:, 0:1] + d1_ref[:, 0:1], 1.0)
        aggr = (p0_ref[...] + p1_ref[...]) / deg
        o = o_ref[...]
        hh = h_ref[...]
        m = jnp.maximum(
            aggr + jnp.dot(o, wr_ref[...], preferred_element_type=jnp.float32)
            + bc_ref[0:1, :], 0.0)
        gi = jnp.dot(m, wi_ref[...], preferred_element_type=jnp.float32) \
            + bi_ref[0:1, :]
        gh = jnp.dot(hh, wh_ref[...], preferred_element_type=jnp.float32) \
            + bh_ref[0:1, :]
        r = jax.nn.sigmoid(gi[:, 0:H] + gh[:, 0:H])
        z = jax.nn.sigmoid(gi[:, H:2 * H] + gh[:, H:2 * H])
        n = jnp.tanh(gi[:, 2 * H:3 * H] + r * gh[:, 2 * H:3 * H])
        new_ref[...] = (1.0 - z) * n + z * hh

    return pl.pallas_call(
        body, out_shape=jax.ShapeDtypeStruct((NP, H), jnp.float32)
    )(p0, p1, d0, d1, out, h, Wroot, bconvb, WihT, WhhT, bihb, bhhb)


def _tc_set2set(out, batch2, WlihT, WlhhT, blib, blhb, W1, b1b, W2, b2b):
    def body(o_ref, b_ref, wli_ref, wlh_ref, bli_ref, blh_ref,
             w1_ref, b1_ref, w2_ref, b2_ref, res_ref):
        o = o_ref[...]
        bcol = b_ref[...]
        seg = lax.broadcasted_iota(jnp.int32, (1, B), 1)
        mask = bcol == seg
        maskf = mask.astype(jnp.float32)
        q_star = jnp.zeros((B, 2 * H), dtype=jnp.float32)
        hl = jnp.zeros((B, H), dtype=jnp.float32)
        cl = jnp.zeros((B, H), dtype=jnp.float32)
        for _ in range(3):
            g = (jnp.dot(q_star, wli_ref[...],
                         preferred_element_type=jnp.float32) + bli_ref[0:1, :]
                 + jnp.dot(hl, wlh_ref[...],
                           preferred_element_type=jnp.float32) + blh_ref[0:1, :])
            ig = jax.nn.sigmoid(g[:, 0:H])
            fg = jax.nn.sigmoid(g[:, H:2 * H])
            gg = jnp.tanh(g[:, 2 * H:3 * H])
            og = jax.nn.sigmoid(g[:, 3 * H:4 * H])
            cl = fg * cl + ig * gg
            hl = og * jnp.tanh(cl)
            q = hl
            qn = jnp.dot(maskf, q, preferred_element_type=jnp.float32)
            e = jnp.sum(o * qn, axis=1, keepdims=True)
            em = jnp.where(mask, e, -jnp.inf)
            emax = jnp.max(em, axis=0, keepdims=True)
            emax = jnp.where(emax == -jnp.inf, 0.0, emax)
            emaxn = jnp.sum(maskf * emax, axis=1, keepdims=True)
            ee = jnp.exp(e - emaxn)
            esum = jnp.sum(maskf * ee, axis=0, keepdims=True)
            esumn = jnp.sum(maskf * esum, axis=1, keepdims=True)
            a = ee / (esumn + 1e-16)
            aw = maskf * a
            r = lax.dot_general(aw, o, (((0,), (0,)), ((), ())),
                                preferred_element_type=jnp.float32)
            q_star = jnp.concatenate([q, r], axis=1)
        z1 = jnp.maximum(
            jnp.dot(q_star, w1_ref[...], preferred_element_type=jnp.float32)
            + b1_ref[0:1, :], 0.0)
        res_ref[...] = jnp.dot(z1, w2_ref[...],
                               preferred_element_type=jnp.float32) + b2_ref[0:1, :]

    return pl.pallas_call(
        body, out_shape=jax.ShapeDtypeStruct((B, OUT), jnp.float32)
    )(out, batch2, WlihT, WlhhT, blib, blhb, W1, b1b, W2, b2b)


def _bias8(b):
    return jnp.broadcast_to(b, (8,) + b.shape)


def kernel(x, edge_index, edge_attr, batch, W0, b0, Wn1, bn1, Wn2, bn2, Wroot,
           bconv, Wih, Whh, bih, bhh, Wl_ih, Wl_hh, bl_ih, bl_hh, W1, b1, W2,
           b2):
    src = edge_index[0]
    dst = edge_index[1]
    xp = jnp.pad(x, ((0, NP - N), (0, 0)))
    srcp = jnp.pad(src, (0, EP - E)).reshape(NW, NCH, CH)
    dstp = jnp.pad(dst, (0, EP - E), constant_values=NP - 1).reshape(NW, NCH, CH)
    eap = jnp.pad(edge_attr, ((0, EP - E), (0, 0)))
    batch2 = jnp.pad(batch, (0, NP - N), constant_values=B).reshape(NP, 1)
    zeros_h = jnp.zeros((NP, H), jnp.float32)
    zeros_d = jnp.zeros((NP, 16), jnp.float32)
    ones_d = jnp.ones((CH, 16), jnp.float32)

    out = _tc_node_init(xp, W0, _bias8(b0))
    d = _sc_degree(dstp, ones_d, zeros_d)
    h = out
    for _ in range(3):
        xj = _sc_gather(out, srcp)
        msg = _tc_msg(eap, xj, Wn1, _bias8(bn1), Wn2, _bias8(bn2))
        p = _sc_scatter_add(msg, dstp, zeros_h)
        h = _tc_update(p[0], p[1], d[0], d[1], out, h, Wroot, _bias8(bconv),
                       Wih.T, Whh.T, _bias8(bih), _bias8(bhh))
        out = h
    return _tc_set2set(out, batch2, Wl_ih.T, Wl_hh.T, _bias8(bl_ih),
                       _bias8(bl_hh), W1, _bias8(b1), W2, _bias8(b2))


# speedup vs baseline: 3.0924x; 3.0924x over previous
import functools

import jax
import jax.numpy as jnp
from jax import lax
from jax.experimental import pallas as pl
from jax.experimental.pallas import tpu as pltpu
from jax.experimental.pallas import tpu_sc as plsc

N = 10000
NP = 10240
E = 160000
EP = 163840
NW = 32
CH = 128
NCH = EP // (NW * CH)
H = 32
ED = 16
B = 64
OUT = 128

@functools.cache
def _mesh():
    return plsc.VectorSubcoreMesh(core_axis_name="c", subcore_axis_name="s")


_SC_PARAMS = pltpu.CompilerParams(use_tc_tiling_on_sc=False)


def _sc_gather(table, idx3):
    @functools.partial(
        pl.kernel, mesh=_mesh(), compiler_params=_SC_PARAMS,
        out_type=jax.ShapeDtypeStruct((EP, H), jnp.float32),
        scratch_types=[
            pltpu.VMEM((NCH, CH), jnp.int32),
            pltpu.VMEM((CH, H), jnp.float32),
            pltpu.VMEM((CH, H), jnp.float32),
            pltpu.SemaphoreType.DMA,
            pltpu.SemaphoreType.DMA,
            pltpu.SemaphoreType.DMA,
            pltpu.SemaphoreType.DMA,
        ],
    )
    def k(table_hbm, idx_hbm, out_hbm, idx_v, buf0, buf1, sg0, sg1, sw0, sw1):
        wid = lax.axis_index("s") * 2 + lax.axis_index("c")
        base = wid * (NCH * CH)
        pltpu.sync_copy(idx_hbm.at[wid], idx_v)
        bufs, gsem, wsem = (buf0, buf1), (sg0, sg1), (sw0, sw1)
        g = [None, None]
        w = [None, None]
        g[0] = pltpu.async_copy(table_hbm.at[idx_v.at[0]], bufs[0], gsem[0])
        for j in range(NCH):
            b = j % 2
            nb = (j + 1) % 2
            if j + 1 < NCH:
                if w[nb] is not None:
                    w[nb].wait()
                g[nb] = pltpu.async_copy(
                    table_hbm.at[idx_v.at[j + 1]], bufs[nb], gsem[nb])
            g[b].wait()
            w[b] = pltpu.async_copy(
                bufs[b], out_hbm.at[pl.ds(base + j * CH, CH)], wsem[b])
        w[0].wait()
        w[1].wait()

    return k(table, idx3)


def _sc_scatter_add(vals, idx3, zeros):
    @functools.partial(
        pl.kernel, mesh=_mesh(), compiler_params=_SC_PARAMS,
        out_type=jax.ShapeDtypeStruct((2, NP, H), jnp.float32),
        scratch_types=[
            pltpu.VMEM((NCH, CH), jnp.int32),
            pltpu.VMEM((CH, H), jnp.float32),
            pltpu.VMEM((CH, H), jnp.float32),
            pltpu.VMEM_SHARED((NP, H), jnp.float32),
            pltpu.SemaphoreType.DMA,
            pltpu.SemaphoreType.DMA,
            pltpu.SemaphoreType.DMA,
            pltpu.SemaphoreType.DMA,
        ],
    )
    def k(v_hbm, idx_hbm, z_hbm, out_hbm, idx_v, buf0, buf1, shared,
          sl0, sl1, ss0, ss1):
        cid = lax.axis_index("c")
        sid = lax.axis_index("s")
        wid = sid * 2 + cid
        rows = NP // 16
        pltpu.sync_copy(z_hbm.at[pl.ds(sid * rows, rows)],
                        shared.at[pl.ds(sid * rows, rows)])
        pltpu.sync_copy(idx_hbm.at[wid], idx_v)
        plsc.subcore_barrier()
        bufs, lsem, ssem = (buf0, buf1), (sl0, sl1), (ss0, ss1)
        ld = [None, None]
        st = [None, None]
        ld[0] = pltpu.async_copy(
            v_hbm.at[pl.ds(wid * (NCH * CH), CH)], bufs[0], lsem[0])
        for j in range(NCH):
            b = j % 2
            nb = (j + 1) % 2
            if j + 1 < NCH:
                if st[nb] is not None:
                    st[nb].wait()
                ld[nb] = pltpu.async_copy(
                    v_hbm.at[pl.ds(wid * (NCH * CH) + (j + 1) * CH, CH)],
                    bufs[nb], lsem[nb])
            ld[b].wait()
            st[b] = pltpu.async_copy(
                bufs[b], shared.at[idx_v.at[j]], ssem[b], add=True)
        st[0].wait()
        st[1].wait()
        plsc.subcore_barrier()
        pltpu.sync_copy(shared.at[pl.ds(sid * rows, rows)],
                        out_hbm.at[cid, pl.ds(sid * rows, rows)])

    return k(vals, idx3, zeros)


def _sc_degree(idx3, ones, zeros):
    @functools.partial(
        pl.kernel, mesh=_mesh(), compiler_params=_SC_PARAMS,
        out_type=jax.ShapeDtypeStruct((2, NP, 16), jnp.float32),
        scratch_types=[
            pltpu.VMEM((NCH, CH), jnp.int32),
            pltpu.VMEM((CH, 16), jnp.float32),
            pltpu.VMEM_SHARED((NP, 16), jnp.float32),
            pltpu.SemaphoreType.DMA,
        ],
    )
    def k(idx_hbm, ones_hbm, z_hbm, out_hbm, idx_v, buf, shared, sem):
        cid = lax.axis_index("c")
        sid = lax.axis_index("s")
        wid = sid * 2 + cid
        rows = NP // 16
        pltpu.sync_copy(z_hbm.at[pl.ds(sid * rows, rows)],
                        shared.at[pl.ds(sid * rows, rows)])
        pltpu.sync_copy(idx_hbm.at[wid], idx_v)
        pltpu.sync_copy(ones_hbm, buf)
        plsc.subcore_barrier()

        @pl.loop(0, NCH)
        def _(j):
            pltpu.sync_copy(buf, shared.at[idx_v.at[j]], add=True)

        plsc.subcore_barrier()
        pltpu.sync_copy(shared.at[pl.ds(sid * rows, rows)],
                        out_hbm.at[cid, pl.ds(sid * rows, rows)])

    return k(idx3, ones, zeros)


def _tc_node_init(xp, W0, b0b):
    def body(x_ref, w_ref, b_ref, o_ref):
        o_ref[...] = jnp.maximum(
            jnp.dot(x_ref[...], w_ref[...], preferred_element_type=jnp.float32)
            + b_ref[0:1, :], 0.0)

    return pl.pallas_call(
        body, out_shape=jax.ShapeDtypeStruct((NP, H), jnp.float32)
    )(xp, W0, b0b)


_TE = 1024


def _tc_msg(eap, xj, Wn1, bn1b, Wn2bf, bn2b, Rk):
    def body(ea_ref, xj_ref, w1_ref, b1_ref, w2_ref, b2_ref, r_ref, o_ref):
        f = jnp.maximum(
            jnp.dot(ea_ref[...], w1_ref[...], preferred_element_type=jnp.float32)
            + b1_ref[0:1, :], 0.0)
        U = jnp.dot(f.astype(jnp.bfloat16), w2_ref[...],
                    preferred_element_type=jnp.float32) + b2_ref[0:1, :]
        xjr = jnp.dot(xj_ref[...], r_ref[...],
                      preferred_element_type=jnp.float32)
        p = xjr * U
        w = H * H
        while w > H:
            w //= 2
            p = p[:, :w] + p[:, w:2 * w]
        o_ref[...] = p

    return pl.pallas_call(
        body,
        grid=(EP // _TE,),
        in_specs=[
            pl.BlockSpec((_TE, ED), lambda i: (i, 0)),
            pl.BlockSpec((_TE, H), lambda i: (i, 0)),
            pl.BlockSpec((ED, 128), lambda i: (0, 0)),
            pl.BlockSpec((8, 128), lambda i: (0, 0)),
            pl.BlockSpec((128, H * H), lambda i: (0, 0)),
            pl.BlockSpec((8, H * H), lambda i: (0, 0)),
            pl.BlockSpec((H, H * H), lambda i: (0, 0)),
        ],
        out_specs=pl.BlockSpec((_TE, H), lambda i: (i, 0)),
        out_shape=jax.ShapeDtypeStruct((EP, H), jnp.float32),
    )(eap, xj, Wn1, bn1b, Wn2bf, bn2b, Rk)


def _tc_update(p0, p1, d0, d1, out, h, Wroot, bconvb, WihT, WhhT, bihb, bhhb):
    def body(p0_ref, p1_ref, d0_ref, d1_ref, o_ref, h_ref, wr_ref, bc_ref,
             wi_ref, wh_ref, bi_ref, bh_ref, new_ref):
        deg = jnp.maximum(d0_ref[:, 0:1] + d1_ref[:, 0:1], 1.0)
        aggr = (p0_ref[...] + p1_ref[...]) / deg
        o = o_ref[...]
        hh = h_ref[...]
        m = jnp.maximum(
            aggr + jnp.dot(o, wr_ref[...], preferred_element_type=jnp.float32)
            + bc_ref[0:1, :], 0.0)
        gi = jnp.dot(m, wi_ref[...], preferred_element_type=jnp.float32) \
            + bi_ref[0:1, :]
        gh = jnp.dot(hh, wh_ref[...], preferred_element_type=jnp.float32) \
            + bh_ref[0:1, :]
        r = jax.nn.sigmoid(gi[:, 0:H] + gh[:, 0:H])
        z = jax.nn.sigmoid(gi[:, H:2 * H] + gh[:, H:2 * H])
        n = jnp.tanh(gi[:, 2 * H:3 * H] + r * gh[:, 2 * H:3 * H])
        new_ref[...] = (1.0 - z) * n + z * hh

    return pl.pallas_call(
        body, out_shape=jax.ShapeDtypeStruct((NP, H), jnp.float32)
    )(p0, p1, d0, d1, out, h, Wroot, bconvb, WihT, WhhT, bihb, bhhb)


def _tc_set2set(out, batch2, WlihT, WlhhT, blib, blhb, W1, b1b, W2, b2b):
    def body(o_ref, b_ref, wli_ref, wlh_ref, bli_ref, blh_ref,
             w1_ref, b1_ref, w2_ref, b2_ref, res_ref):
        o = o_ref[...]
        bcol = b_ref[...]
        seg = lax.broadcasted_iota(jnp.int32, (1, B), 1)
        mask = bcol == seg
        maskf = mask.astype(jnp.float32)
        q_star = jnp.zeros((B, 2 * H), dtype=jnp.float32)
        hl = jnp.zeros((B, H), dtype=jnp.float32)
        cl = jnp.zeros((B, H), dtype=jnp.float32)
        for _ in range(3):
            g = (jnp.dot(q_star, wli_ref[...],
                         preferred_element_type=jnp.float32) + bli_ref[0:1, :]
                 + jnp.dot(hl, wlh_ref[...],
                           preferred_element_type=jnp.float32) + blh_ref[0:1, :])
            ig = jax.nn.sigmoid(g[:, 0:H])
            fg = jax.nn.sigmoid(g[:, H:2 * H])
            gg = jnp.tanh(g[:, 2 * H:3 * H])
            og = jax.nn.sigmoid(g[:, 3 * H:4 * H])
            cl = fg * cl + ig * gg
            hl = og * jnp.tanh(cl)
            q = hl
            qn = jnp.dot(maskf, q, preferred_element_type=jnp.float32)
            e = jnp.sum(o * qn, axis=1, keepdims=True)
            em = jnp.where(mask, e, -jnp.inf)
            emax = jnp.max(em, axis=0, keepdims=True)
            emax = jnp.where(emax == -jnp.inf, 0.0, emax)
            emaxn = jnp.sum(maskf * emax, axis=1, keepdims=True)
            ee = jnp.exp(e - emaxn)
            esum = jnp.sum(maskf * ee, axis=0, keepdims=True)
            esumn = jnp.sum(maskf * esum, axis=1, keepdims=True)
            a = ee / (esumn + 1e-16)
            aw = maskf * a
            r = lax.dot_general(aw, o, (((0,), (0,)), ((), ())),
                                preferred_element_type=jnp.float32)
            q_star = jnp.concatenate([q, r], axis=1)
        z1 = jnp.maximum(
            jnp.dot(q_star, w1_ref[...], preferred_element_type=jnp.float32)
            + b1_ref[0:1, :], 0.0)
        res_ref[...] = jnp.dot(z1, w2_ref[...],
                               preferred_element_type=jnp.float32) + b2_ref[0:1, :]

    return pl.pallas_call(
        body, out_shape=jax.ShapeDtypeStruct((B, OUT), jnp.float32)
    )(out, batch2, WlihT, WlhhT, blib, blhb, W1, b1b, W2, b2b)


def _bias8(b):
    return jnp.broadcast_to(b, (8,) + b.shape)


def kernel(x, edge_index, edge_attr, batch, W0, b0, Wn1, bn1, Wn2, bn2, Wroot,
           bconv, Wih, Whh, bih, bhh, Wl_ih, Wl_hh, bl_ih, bl_hh, W1, b1, W2,
           b2):
    src = edge_index[0]
    dst = edge_index[1]
    xp = jnp.pad(x, ((0, NP - N), (0, 0)))
    srcp = jnp.pad(src, (0, EP - E)).reshape(NW, NCH, CH)
    dstp = jnp.pad(dst, (0, EP - E), constant_values=NP - 1).reshape(NW, NCH, CH)
    eap = jnp.pad(edge_attr, ((0, EP - E), (0, 0)))
    batch2 = jnp.pad(batch, (0, NP - N), constant_values=B).reshape(NP, 1)
    zeros_h = jnp.zeros((NP, H), jnp.float32)
    zeros_d = jnp.zeros((NP, 16), jnp.float32)
    ones_d = jnp.ones((CH, 16), jnp.float32)
    Wn2bf = Wn2.astype(jnp.bfloat16)
    Rk = jnp.repeat(jnp.eye(H, dtype=jnp.float32), H, axis=1)

    out = _tc_node_init(xp, W0, _bias8(b0))
    d = _sc_degree(dstp, ones_d, zeros_d)
    h = out
    for _ in range(3):
        xj = _sc_gather(out, srcp)
        msg = _tc_msg(eap, xj, Wn1, _bias8(bn1), Wn2bf, _bias8(bn2), Rk)
        p = _sc_scatter_add(msg, dstp, zeros_h)
        h = _tc_update(p[0], p[1], d[0], d[1], out, h, Wroot, _bias8(bconv),
                       Wih.T, Whh.T, _bias8(bih), _bias8(bhh))
        out = h
    return _tc_set2set(out, batch2, Wl_ih.T, Wl_hh.T, _bias8(bl_ih),
                       _bias8(bl_hh), W1, _bias8(b1), W2, _bias8(b2))


# speedup vs baseline: 3.2726x; 1.0583x over previous
import functools

import jax
import jax.numpy as jnp
from jax import lax
from jax.experimental import pallas as pl
from jax.experimental.pallas import tpu as pltpu
from jax.experimental.pallas import tpu_sc as plsc

N = 10000
NP = 10240
E = 160000
EP = 163840
NW = 32
CH = 128
NCH = EP // (NW * CH)
H = 32
ED = 16
B = 64
OUT = 128

@functools.cache
def _mesh():
    return plsc.VectorSubcoreMesh(core_axis_name="c", subcore_axis_name="s")


_SC_PARAMS = pltpu.CompilerParams(use_tc_tiling_on_sc=False)


def _sc_gather(table, idx3):
    @functools.partial(
        pl.kernel, mesh=_mesh(), compiler_params=_SC_PARAMS,
        out_type=jax.ShapeDtypeStruct((EP, H), jnp.float32),
        scratch_types=[
            pltpu.VMEM((NCH, CH), jnp.int32),
            pltpu.VMEM((CH, H), jnp.float32),
            pltpu.VMEM((CH, H), jnp.float32),
            pltpu.SemaphoreType.DMA,
            pltpu.SemaphoreType.DMA,
            pltpu.SemaphoreType.DMA,
            pltpu.SemaphoreType.DMA,
        ],
    )
    def k(table_hbm, idx_hbm, out_hbm, idx_v, buf0, buf1, sg0, sg1, sw0, sw1):
        wid = lax.axis_index("s") * 2 + lax.axis_index("c")
        base = wid * (NCH * CH)
        pltpu.sync_copy(idx_hbm.at[wid], idx_v)
        bufs, gsem, wsem = (buf0, buf1), (sg0, sg1), (sw0, sw1)
        g = [None, None]
        w = [None, None]
        g[0] = pltpu.async_copy(table_hbm.at[idx_v.at[0]], bufs[0], gsem[0])
        for j in range(NCH):
            b = j % 2
            nb = (j + 1) % 2
            if j + 1 < NCH:
                if w[nb] is not None:
                    w[nb].wait()
                g[nb] = pltpu.async_copy(
                    table_hbm.at[idx_v.at[j + 1]], bufs[nb], gsem[nb])
            g[b].wait()
            w[b] = pltpu.async_copy(
                bufs[b], out_hbm.at[pl.ds(base + j * CH, CH)], wsem[b])
        w[0].wait()
        w[1].wait()

    return k(table, idx3)


def _sc_scatter_add(vals, idx3, zeros):
    @functools.partial(
        pl.kernel, mesh=_mesh(), compiler_params=_SC_PARAMS,
        out_type=jax.ShapeDtypeStruct((2, NP, H), jnp.float32),
        scratch_types=[
            pltpu.VMEM((NCH, CH), jnp.int32),
            pltpu.VMEM((CH, H), jnp.float32),
            pltpu.VMEM((CH, H), jnp.float32),
            pltpu.VMEM_SHARED((NP, H), jnp.float32),
            pltpu.SemaphoreType.DMA,
            pltpu.SemaphoreType.DMA,
            pltpu.SemaphoreType.DMA,
            pltpu.SemaphoreType.DMA,
        ],
    )
    def k(v_hbm, idx_hbm, z_hbm, out_hbm, idx_v, buf0, buf1, shared,
          sl0, sl1, ss0, ss1):
        cid = lax.axis_index("c")
        sid = lax.axis_index("s")
        wid = sid * 2 + cid
        rows = NP // 16
        pltpu.sync_copy(z_hbm.at[pl.ds(sid * rows, rows)],
                        shared.at[pl.ds(sid * rows, rows)])
        pltpu.sync_copy(idx_hbm.at[wid], idx_v)
        plsc.subcore_barrier()
        bufs, lsem, ssem = (buf0, buf1), (sl0, sl1), (ss0, ss1)
        ld = [None, None]
        st = [None, None]
        ld[0] = pltpu.async_copy(
            v_hbm.at[pl.ds(wid * (NCH * CH), CH)], bufs[0], lsem[0])
        for j in range(NCH):
            b = j % 2
            nb = (j + 1) % 2
            if j + 1 < NCH:
                if st[nb] is not None:
                    st[nb].wait()
                ld[nb] = pltpu.async_copy(
                    v_hbm.at[pl.ds(wid * (NCH * CH) + (j + 1) * CH, CH)],
                    bufs[nb], lsem[nb])
            ld[b].wait()
            st[b] = pltpu.async_copy(
                bufs[b], shared.at[idx_v.at[j]], ssem[b], add=True)
        st[0].wait()
        st[1].wait()
        plsc.subcore_barrier()
        pltpu.sync_copy(shared.at[pl.ds(sid * rows, rows)],
                        out_hbm.at[cid, pl.ds(sid * rows, rows)])

    return k(vals, idx3, zeros)


def _sc_degree(idx3, ones, zeros):
    @functools.partial(
        pl.kernel, mesh=_mesh(), compiler_params=_SC_PARAMS,
        out_type=jax.ShapeDtypeStruct((2, NP, 16), jnp.float32),
        scratch_types=[
            pltpu.VMEM((NCH, CH), jnp.int32),
            pltpu.VMEM((CH, 16), jnp.float32),
            pltpu.VMEM_SHARED((NP, 16), jnp.float32),
            pltpu.SemaphoreType.DMA,
        ],
    )
    def k(idx_hbm, ones_hbm, z_hbm, out_hbm, idx_v, buf, shared, sem):
        cid = lax.axis_index("c")
        sid = lax.axis_index("s")
        wid = sid * 2 + cid
        rows = NP // 16
        pltpu.sync_copy(z_hbm.at[pl.ds(sid * rows, rows)],
                        shared.at[pl.ds(sid * rows, rows)])
        pltpu.sync_copy(idx_hbm.at[wid], idx_v)
        pltpu.sync_copy(ones_hbm, buf)
        plsc.subcore_barrier()

        @pl.loop(0, NCH)
        def _(j):
            pltpu.sync_copy(buf, shared.at[idx_v.at[j]], add=True)

        plsc.subcore_barrier()
        pltpu.sync_copy(shared.at[pl.ds(sid * rows, rows)],
                        out_hbm.at[cid, pl.ds(sid * rows, rows)])

    return k(idx3, ones, zeros)


def _tc_node_init(xp, W0, b0b):
    def body(x_ref, w_ref, b_ref, o_ref):
        o_ref[...] = jnp.maximum(
            jnp.dot(x_ref[...], w_ref[...], preferred_element_type=jnp.float32)
            + b_ref[0:1, :], 0.0)

    return pl.pallas_call(
        body, out_shape=jax.ShapeDtypeStruct((NP, H), jnp.float32)
    )(xp, W0, b0b)


_TE = 2048


def _tc_msg(eap, xj, Wn1, bn1b, Wn2bf, bn2b, Rk):
    def body(ea_ref, xj_ref, w1_ref, b1_ref, w2_ref, b2_ref, r_ref, o_ref):
        f = jnp.maximum(
            jnp.dot(ea_ref[...], w1_ref[...], preferred_element_type=jnp.float32)
            + b1_ref[0:1, :], 0.0)
        U = jnp.dot(f.astype(jnp.bfloat16), w2_ref[...],
                    preferred_element_type=jnp.float32) + b2_ref[0:1, :]
        xjr = jnp.dot(xj_ref[...].astype(jnp.bfloat16), r_ref[...],
                      preferred_element_type=jnp.float32)
        p = xjr * U
        w = H * H
        while w > H:
            w //= 2
            p = p[:, :w] + p[:, w:2 * w]
        o_ref[...] = p

    return pl.pallas_call(
        body,
        grid=(EP // _TE,),
        in_specs=[
            pl.BlockSpec((_TE, ED), lambda i: (i, 0)),
            pl.BlockSpec((_TE, H), lambda i: (i, 0)),
            pl.BlockSpec((ED, 128), lambda i: (0, 0)),
            pl.BlockSpec((8, 128), lambda i: (0, 0)),
            pl.BlockSpec((128, H * H), lambda i: (0, 0)),
            pl.BlockSpec((8, H * H), lambda i: (0, 0)),
            pl.BlockSpec((H, H * H), lambda i: (0, 0)),
        ],
        out_specs=pl.BlockSpec((_TE, H), lambda i: (i, 0)),
        out_shape=jax.ShapeDtypeStruct((EP, H), jnp.float32),
    )(eap, xj, Wn1, bn1b, Wn2bf, bn2b, Rk)


def _tc_update(p0, p1, d0, d1, out, h, Wroot, bconvb, WihT, WhhT, bihb, bhhb):
    def body(p0_ref, p1_ref, d0_ref, d1_ref, o_ref, h_ref, wr_ref, bc_ref,
             wi_ref, wh_ref, bi_ref, bh_ref, new_ref):
        deg = jnp.maximum(d0_ref[:, 0:1] + d1_ref[:, 0:1], 1.0)
        aggr = (p0_ref[...] + p1_ref[...]) / deg
        o = o_ref[...]
        hh = h_ref[...]
        m = jnp.maximum(
            aggr + jnp.dot(o, wr_ref[...], preferred_element_type=jnp.float32)
            + bc_ref[0:1, :], 0.0)
        gi = jnp.dot(m, wi_ref[...], preferred_element_type=jnp.float32) \
            + bi_ref[0:1, :]
        gh = jnp.dot(hh, wh_ref[...], preferred_element_type=jnp.float32) \
            + bh_ref[0:1, :]
        r = jax.nn.sigmoid(gi[:, 0:H] + gh[:, 0:H])
        z = jax.nn.sigmoid(gi[:, H:2 * H] + gh[:, H:2 * H])
        n = jnp.tanh(gi[:, 2 * H:3 * H] + r * gh[:, 2 * H:3 * H])
        new_ref[...] = (1.0 - z) * n + z * hh

    return pl.pallas_call(
        body, out_shape=jax.ShapeDtypeStruct((NP, H), jnp.float32)
    )(p0, p1, d0, d1, out, h, Wroot, bconvb, WihT, WhhT, bihb, bhhb)


def _tc_set2set(out, batch2, WlihT, WlhhT, blib, blhb, W1, b1b, W2, b2b):
    def body(o_ref, b_ref, wli_ref, wlh_ref, bli_ref, blh_ref,
             w1_ref, b1_ref, w2_ref, b2_ref, res_ref):
        o = o_ref[...]
        bcol = b_ref[...]
        seg = lax.broadcasted_iota(jnp.int32, (1, B), 1)
        mask = bcol == seg
        maskf = mask.astype(jnp.float32)
        q_star = jnp.zeros((B, 2 * H), dtype=jnp.float32)
        hl = jnp.zeros((B, H), dtype=jnp.float32)
        cl = jnp.zeros((B, H), dtype=jnp.float32)
        for _ in range(3):
            g = (jnp.dot(q_star, wli_ref[...],
                         preferred_element_type=jnp.float32) + bli_ref[0:1, :]
                 + jnp.dot(hl, wlh_ref[...],
                           preferred_element_type=jnp.float32) + blh_ref[0:1, :])
            ig = jax.nn.sigmoid(g[:, 0:H])
            fg = jax.nn.sigmoid(g[:, H:2 * H])
            gg = jnp.tanh(g[:, 2 * H:3 * H])
            og = jax.nn.sigmoid(g[:, 3 * H:4 * H])
            cl = fg * cl + ig * gg
            hl = og * jnp.tanh(cl)
            q = hl
            qn = jnp.dot(maskf, q, preferred_element_type=jnp.float32)
            e = jnp.sum(o * qn, axis=1, keepdims=True)
            em = jnp.where(mask, e, -jnp.inf)
            emax = jnp.max(em, axis=0, keepdims=True)
            emax = jnp.where(emax == -jnp.inf, 0.0, emax)
            emaxn = jnp.sum(maskf * emax, axis=1, keepdims=True)
            ee = jnp.exp(e - emaxn)
            esum = jnp.sum(maskf * ee, axis=0, keepdims=True)
            esumn = jnp.sum(maskf * esum, axis=1, keepdims=True)
            a = ee / (esumn + 1e-16)
            aw = maskf * a
            r = lax.dot_general(aw, o, (((0,), (0,)), ((), ())),
                                preferred_element_type=jnp.float32)
            q_star = jnp.concatenate([q, r], axis=1)
        z1 = jnp.maximum(
            jnp.dot(q_star, w1_ref[...], preferred_element_type=jnp.float32)
            + b1_ref[0:1, :], 0.0)
        res_ref[...] = jnp.dot(z1, w2_ref[...],
                               preferred_element_type=jnp.float32) + b2_ref[0:1, :]

    return pl.pallas_call(
        body, out_shape=jax.ShapeDtypeStruct((B, OUT), jnp.float32)
    )(out, batch2, WlihT, WlhhT, blib, blhb, W1, b1b, W2, b2b)


def _bias8(b):
    return jnp.broadcast_to(b, (8,) + b.shape)


def kernel(x, edge_index, edge_attr, batch, W0, b0, Wn1, bn1, Wn2, bn2, Wroot,
           bconv, Wih, Whh, bih, bhh, Wl_ih, Wl_hh, bl_ih, bl_hh, W1, b1, W2,
           b2):
    src = edge_index[0]
    dst = edge_index[1]
    xp = jnp.pad(x, ((0, NP - N), (0, 0)))
    srcp = jnp.pad(src, (0, EP - E)).reshape(NW, NCH, CH)
    dstp = jnp.pad(dst, (0, EP - E), constant_values=NP - 1).reshape(NW, NCH, CH)
    eap = jnp.pad(edge_attr, ((0, EP - E), (0, 0)))
    batch2 = jnp.pad(batch, (0, NP - N), constant_values=B).reshape(NP, 1)
    zeros_h = jnp.zeros((NP, H), jnp.float32)
    zeros_d = jnp.zeros((NP, 16), jnp.float32)
    ones_d = jnp.ones((CH, 16), jnp.float32)
    Wn2bf = Wn2.astype(jnp.bfloat16)
    Rk = jnp.repeat(jnp.eye(H, dtype=jnp.bfloat16), H, axis=1)

    out = _tc_node_init(xp, W0, _bias8(b0))
    d = _sc_degree(dstp, ones_d, zeros_d)
    h = out
    for _ in range(3):
        xj = _sc_gather(out, srcp)
        msg = _tc_msg(eap, xj, Wn1, _bias8(bn1), Wn2bf, _bias8(bn2), Rk)
        p = _sc_scatter_add(msg, dstp, zeros_h)
        h = _tc_update(p[0], p[1], d[0], d[1], out, h, Wroot, _bias8(bconv),
                       Wih.T, Whh.T, _bias8(bih), _bias8(bhh))
        out = h
    return _tc_set2set(out, batch2, Wl_ih.T, Wl_hh.T, _bias8(bl_ih),
                       _bias8(bl_hh), W1, _bias8(b1), W2, _bias8(b2))


# speedup vs baseline: 3.3363x; 1.0195x over previous
import functools

import jax
import jax.numpy as jnp
from jax import lax
from jax.experimental import pallas as pl
from jax.experimental.pallas import tpu as pltpu
from jax.experimental.pallas import tpu_sc as plsc

N = 10000
NP = 10240
E = 160000
EP = 163840
NW = 32
CH = 128
NCH = EP // (NW * CH)
H = 32
ED = 16
B = 64
OUT = 128

@functools.cache
def _mesh():
    return plsc.VectorSubcoreMesh(core_axis_name="c", subcore_axis_name="s")


_SC_PARAMS = pltpu.CompilerParams(use_tc_tiling_on_sc=False)


def _sc_gather(table, idx3):
    @functools.partial(
        pl.kernel, mesh=_mesh(), compiler_params=_SC_PARAMS,
        out_type=jax.ShapeDtypeStruct((EP, H), jnp.float32),
        scratch_types=[
            pltpu.VMEM((NCH, CH), jnp.int32),
            pltpu.VMEM((CH, H), jnp.float32),
            pltpu.VMEM((CH, H), jnp.float32),
            pltpu.VMEM((CH, H), jnp.float32),
            pltpu.VMEM((CH, H), jnp.float32),
            pltpu.SemaphoreType.DMA,
            pltpu.SemaphoreType.DMA,
            pltpu.SemaphoreType.DMA,
            pltpu.SemaphoreType.DMA,
            pltpu.SemaphoreType.DMA,
            pltpu.SemaphoreType.DMA,
            pltpu.SemaphoreType.DMA,
            pltpu.SemaphoreType.DMA,
        ],
    )
    def k(table_hbm, idx_hbm, out_hbm, idx_v,
          buf0, buf1, buf2, buf3, sg0, sg1, sg2, sg3, sw0, sw1, sw2, sw3):
        wid = lax.axis_index("s") * 2 + lax.axis_index("c")
        base = wid * (NCH * CH)
        pltpu.sync_copy(idx_hbm.at[wid], idx_v)
        nb_ = 4
        bufs = (buf0, buf1, buf2, buf3)
        gsem = (sg0, sg1, sg2, sg3)
        wsem = (sw0, sw1, sw2, sw3)
        g = [None] * nb_
        w = [None] * nb_
        for j in range(nb_):
            g[j] = pltpu.async_copy(table_hbm.at[idx_v.at[j]], bufs[j], gsem[j])
        for j in range(NCH):
            b = j % nb_
            g[b].wait()
            w[b] = pltpu.async_copy(
                bufs[b], out_hbm.at[pl.ds(base + j * CH, CH)], wsem[b])
            jn = j + nb_
            if jn < NCH:
                w[b].wait()
                g[b] = pltpu.async_copy(
                    table_hbm.at[idx_v.at[jn]], bufs[b], gsem[b])
        for b in range(nb_):
            w[b].wait()

    return k(table, idx3)


def _sc_scatter_add(vals, idx3, zeros):
    @functools.partial(
        pl.kernel, mesh=_mesh(), compiler_params=_SC_PARAMS,
        out_type=jax.ShapeDtypeStruct((2, NP, H), jnp.float32),
        scratch_types=[
            pltpu.VMEM((NCH, CH), jnp.int32),
            pltpu.VMEM((CH, H), jnp.float32),
            pltpu.VMEM((CH, H), jnp.float32),
            pltpu.VMEM((CH, H), jnp.float32),
            pltpu.VMEM((CH, H), jnp.float32),
            pltpu.VMEM_SHARED((NP, H), jnp.float32),
            pltpu.SemaphoreType.DMA,
            pltpu.SemaphoreType.DMA,
            pltpu.SemaphoreType.DMA,
            pltpu.SemaphoreType.DMA,
            pltpu.SemaphoreType.DMA,
            pltpu.SemaphoreType.DMA,
            pltpu.SemaphoreType.DMA,
            pltpu.SemaphoreType.DMA,
        ],
    )
    def k(v_hbm, idx_hbm, z_hbm, out_hbm, idx_v, buf0, buf1, buf2, buf3,
          shared, sl0, sl1, sl2, sl3, ss0, ss1, ss2, ss3):
        cid = lax.axis_index("c")
        sid = lax.axis_index("s")
        wid = sid * 2 + cid
        rows = NP // 16
        pltpu.sync_copy(z_hbm.at[pl.ds(sid * rows, rows)],
                        shared.at[pl.ds(sid * rows, rows)])
        pltpu.sync_copy(idx_hbm.at[wid], idx_v)
        plsc.subcore_barrier()
        nb_ = 4
        bufs = (buf0, buf1, buf2, buf3)
        lsem = (sl0, sl1, sl2, sl3)
        ssem = (ss0, ss1, ss2, ss3)
        ld = [None] * nb_
        st = [None] * nb_
        for j in range(nb_):
            ld[j] = pltpu.async_copy(
                v_hbm.at[pl.ds(wid * (NCH * CH) + j * CH, CH)],
                bufs[j], lsem[j])
        for j in range(NCH):
            b = j % nb_
            ld[b].wait()
            st[b] = pltpu.async_copy(
                bufs[b], shared.at[idx_v.at[j]], ssem[b], add=True)
            jn = j + nb_
            if jn < NCH:
                st[b].wait()
                ld[b] = pltpu.async_copy(
                    v_hbm.at[pl.ds(wid * (NCH * CH) + jn * CH, CH)],
                    bufs[b], lsem[b])
        for b in range(nb_):
            st[b].wait()
        plsc.subcore_barrier()
        pltpu.sync_copy(shared.at[pl.ds(sid * rows, rows)],
                        out_hbm.at[cid, pl.ds(sid * rows, rows)])

    return k(vals, idx3, zeros)


def _sc_degree(idx3, ones, zeros):
    @functools.partial(
        pl.kernel, mesh=_mesh(), compiler_params=_SC_PARAMS,
        out_type=jax.ShapeDtypeStruct((2, NP, 16), jnp.float32),
        scratch_types=[
            pltpu.VMEM((NCH, CH), jnp.int32),
            pltpu.VMEM((CH, 16), jnp.float32),
            pltpu.VMEM_SHARED((NP, 16), jnp.float32),
            pltpu.SemaphoreType.DMA,
        ],
    )
    def k(idx_hbm, ones_hbm, z_hbm, out_hbm, idx_v, buf, shared, sem):
        cid = lax.axis_index("c")
        sid = lax.axis_index("s")
        wid = sid * 2 + cid
        rows = NP // 16
        pltpu.sync_copy(z_hbm.at[pl.ds(sid * rows, rows)],
                        shared.at[pl.ds(sid * rows, rows)])
        pltpu.sync_copy(idx_hbm.at[wid], idx_v)
        pltpu.sync_copy(ones_hbm, buf)
        plsc.subcore_barrier()

        @pl.loop(0, NCH)
        def _(j):
            pltpu.sync_copy(buf, shared.at[idx_v.at[j]], add=True)

        plsc.subcore_barrier()
        pltpu.sync_copy(shared.at[pl.ds(sid * rows, rows)],
                        out_hbm.at[cid, pl.ds(sid * rows, rows)])

    return k(idx3, ones, zeros)


def _tc_node_init(xp, W0, b0b):
    def body(x_ref, w_ref, b_ref, o_ref):
        o_ref[...] = jnp.maximum(
            jnp.dot(x_ref[...], w_ref[...], preferred_element_type=jnp.float32)
            + b_ref[0:1, :], 0.0)

    return pl.pallas_call(
        body, out_shape=jax.ShapeDtypeStruct((NP, H), jnp.float32)
    )(xp, W0, b0b)


_TE = 2048


def _tc_msg(eap, xj, Wn1, bn1b, Wn2bf, bn2b, Rk):
    def body(ea_ref, xj_ref, w1_ref, b1_ref, w2_ref, b2_ref, r_ref, o_ref):
        f = jnp.maximum(
            jnp.dot(ea_ref[...], w1_ref[...], preferred_element_type=jnp.float32)
            + b1_ref[0:1, :], 0.0)
        U = jnp.dot(f.astype(jnp.bfloat16), w2_ref[...],
                    preferred_element_type=jnp.float32)
        xjb = xj_ref[...].astype(jnp.bfloat16)
        xjr = jnp.dot(xjb, r_ref[...],
                      preferred_element_type=jnp.float32)
        p = xjr * U
        w = H * H
        while w > H:
            w //= 2
            p = p[:, :w] + p[:, w:2 * w]
        o_ref[...] = p + jnp.dot(xjb, b2_ref[...],
                                 preferred_element_type=jnp.float32)

    return pl.pallas_call(
        body,
        grid=(EP // _TE,),
        in_specs=[
            pl.BlockSpec((_TE, ED), lambda i: (i, 0)),
            pl.BlockSpec((_TE, H), lambda i: (i, 0)),
            pl.BlockSpec((ED, 128), lambda i: (0, 0)),
            pl.BlockSpec((8, 128), lambda i: (0, 0)),
            pl.BlockSpec((128, H * H), lambda i: (0, 0)),
            pl.BlockSpec((H, H), lambda i: (0, 0)),
            pl.BlockSpec((H, H * H), lambda i: (0, 0)),
        ],
        out_specs=pl.BlockSpec((_TE, H), lambda i: (i, 0)),
        out_shape=jax.ShapeDtypeStruct((EP, H), jnp.float32),
    )(eap, xj, Wn1, bn1b, Wn2bf, bn2b, Rk)


def _tc_update(p0, p1, d0, d1, out, h, Wroot, bconvb, WihT, WhhT, bihb, bhhb):
    def body(p0_ref, p1_ref, d0_ref, d1_ref, o_ref, h_ref, wr_ref, bc_ref,
             wi_ref, wh_ref, bi_ref, bh_ref, new_ref):
        deg = jnp.maximum(d0_ref[:, 0:1] + d1_ref[:, 0:1], 1.0)
        aggr = (p0_ref[...] + p1_ref[...]) / deg
        o = o_ref[...]
        hh = h_ref[...]
        m = jnp.maximum(
            aggr + jnp.dot(o, wr_ref[...], preferred_element_type=jnp.float32)
            + bc_ref[0:1, :], 0.0)
        gi = jnp.dot(m, wi_ref[...], preferred_element_type=jnp.float32) \
            + bi_ref[0:1, :]
        gh = jnp.dot(hh, wh_ref[...], preferred_element_type=jnp.float32) \
            + bh_ref[0:1, :]
        r = jax.nn.sigmoid(gi[:, 0:H] + gh[:, 0:H])
        z = jax.nn.sigmoid(gi[:, H:2 * H] + gh[:, H:2 * H])
        n = jnp.tanh(gi[:, 2 * H:3 * H] + r * gh[:, 2 * H:3 * H])
        new_ref[...] = (1.0 - z) * n + z * hh

    return pl.pallas_call(
        body, out_shape=jax.ShapeDtypeStruct((NP, H), jnp.float32)
    )(p0, p1, d0, d1, out, h, Wroot, bconvb, WihT, WhhT, bihb, bhhb)


def _tc_set2set(out, batch2, WlihT, WlhhT, blib, blhb, W1, b1b, W2, b2b):
    def body(o_ref, b_ref, wli_ref, wlh_ref, bli_ref, blh_ref,
             w1_ref, b1_ref, w2_ref, b2_ref, res_ref):
        o = o_ref[...]
        bcol = b_ref[...]
        seg = lax.broadcasted_iota(jnp.int32, (1, B), 1)
        mask = bcol == seg
        maskf = mask.astype(jnp.float32)
        q_star = jnp.zeros((B, 2 * H), dtype=jnp.float32)
        hl = jnp.zeros((B, H), dtype=jnp.float32)
        cl = jnp.zeros((B, H), dtype=jnp.float32)
        for _ in range(3):
            g = (jnp.dot(q_star, wli_ref[...],
                         preferred_element_type=jnp.float32) + bli_ref[0:1, :]
                 + jnp.dot(hl, wlh_ref[...],
                           preferred_element_type=jnp.float32) + blh_ref[0:1, :])
            ig = jax.nn.sigmoid(g[:, 0:H])
            fg = jax.nn.sigmoid(g[:, H:2 * H])
            gg = jnp.tanh(g[:, 2 * H:3 * H])
            og = jax.nn.sigmoid(g[:, 3 * H:4 * H])
            cl = fg * cl + ig * gg
            hl = og * jnp.tanh(cl)
            q = hl
            qn = jnp.dot(maskf, q, preferred_element_type=jnp.float32)
            e = jnp.sum(o * qn, axis=1, keepdims=True)
            em = jnp.where(mask, e, -jnp.inf)
            emax = jnp.max(em, axis=0, keepdims=True)
            emax = jnp.where(emax == -jnp.inf, 0.0, emax)
            emaxn = jnp.sum(maskf * emax, axis=1, keepdims=True)
            ee = jnp.exp(e - emaxn)
            esum = jnp.sum(maskf * ee, axis=0, keepdims=True)
            esumn = jnp.sum(maskf * esum, axis=1, keepdims=True)
            a = ee / (esumn + 1e-16)
            aw = maskf * a
            r = lax.dot_general(aw, o, (((0,), (0,)), ((), ())),
                                preferred_element_type=jnp.float32)
            q_star = jnp.concatenate([q, r], axis=1)
        z1 = jnp.maximum(
            jnp.dot(q_star, w1_ref[...], preferred_element_type=jnp.float32)
            + b1_ref[0:1, :], 0.0)
        res_ref[...] = jnp.dot(z1, w2_ref[...],
                               preferred_element_type=jnp.float32) + b2_ref[0:1, :]

    return pl.pallas_call(
        body, out_shape=jax.ShapeDtypeStruct((B, OUT), jnp.float32)
    )(out, batch2, WlihT, WlhhT, blib, blhb, W1, b1b, W2, b2b)


def _bias8(b):
    return jnp.broadcast_to(b, (8,) + b.shape)


def kernel(x, edge_index, edge_attr, batch, W0, b0, Wn1, bn1, Wn2, bn2, Wroot,
           bconv, Wih, Whh, bih, bhh, Wl_ih, Wl_hh, bl_ih, bl_hh, W1, b1, W2,
           b2):
    src = edge_index[0]
    dst = edge_index[1]
    xp = jnp.pad(x, ((0, NP - N), (0, 0)))
    srcp = jnp.pad(src, (0, EP - E)).reshape(NW, NCH, CH)
    dstp = jnp.pad(dst, (0, EP - E), constant_values=NP - 1).reshape(NW, NCH, CH)
    eap = jnp.pad(edge_attr, ((0, EP - E), (0, 0)))
    batch2 = jnp.pad(batch, (0, NP - N), constant_values=B).reshape(NP, 1)
    zeros_h = jnp.zeros((NP, H), jnp.float32)
    zeros_d = jnp.zeros((NP, 16), jnp.float32)
    ones_d = jnp.ones((CH, 16), jnp.float32)
    Wn2bf = Wn2.astype(jnp.bfloat16)
    Bn2bf = bn2.reshape(H, H).astype(jnp.bfloat16)
    Rk = jnp.repeat(jnp.eye(H, dtype=jnp.bfloat16), H, axis=1)

    out = _tc_node_init(xp, W0, _bias8(b0))
    d = _sc_degree(dstp, ones_d, zeros_d)
    h = out
    for _ in range(3):
        xj = _sc_gather(out, srcp)
        msg = _tc_msg(eap, xj, Wn1, _bias8(bn1), Wn2bf, Bn2bf, Rk)
        p = _sc_scatter_add(msg, dstp, zeros_h)
        h = _tc_update(p[0], p[1], d[0], d[1], out, h, Wroot, _bias8(bconv),
                       Wih.T, Whh.T, _bias8(bih), _bias8(bhh))
        out = h
    return _tc_set2set(out, batch2, Wl_ih.T, Wl_hh.T, _bias8(bl_ih),
                       _bias8(bl_hh), W1, _bias8(b1), W2, _bias8(b2))
